# baseline (speedup 1.0000x reference)
import functools

import jax
import jax.numpy as jnp
from jax.experimental import pallas as pl
from jax.experimental.pallas import tpu as pltpu

N = 4096
D = 512
TOPK = 32
INT_MIN = -2147483648


def _topk_norm_kernel(w_ref, out_ref):
    R = w_ref.shape[0]
    w = w_ref[...]
    key = jax.lax.bitcast_convert_type(w, jnp.int32)
    key = key ^ (jax.lax.shift_right_arithmetic(key, 31) & jnp.int32(0x7FFFFFFF))
    hi = jax.lax.shift_right_arithmetic(key, 16).astype(jnp.int16)

    def _count16(mask):
        mbf = jnp.where(mask, jnp.bfloat16(1), jnp.bfloat16(0))
        p = jnp.sum(mbf.reshape(R, N // 128, 128), axis=1)
        return jnp.sum(p.astype(jnp.float32), axis=1, keepdims=True)

    def _count32(mask):
        mf = jnp.where(mask, 1.0, 0.0)
        p = jnp.sum(mf.reshape(R, N // 128, 128), axis=1)
        return jnp.sum(p, axis=1, keepdims=True)

    def bit_body16(i, t_y):
        b = 15 - i
        cand_y = t_y | jnp.left_shift(jnp.int32(1), b)
        cand_key = (cand_y ^ 0x8000).astype(jnp.int16)
        cnt = _count16(hi >= cand_key)
        return jnp.where(cnt >= TOPK, cand_y, t_y)

    t_y16 = jax.lax.fori_loop(0, 16, bit_body16, jnp.zeros((R, 1), jnp.int32))

    t16 = (t_y16 ^ 0x8000).astype(jnp.int16)
    gt_hi = hi > t16
    eq_hi = hi == t16
    cnt_gt_hi = _count16(gt_hi)
    lo_b = ((key & 0xFFFF) ^ 0x8000).astype(jnp.int16)

    def bit_body_lo(i, t_lo_y):
        b = 15 - i
        cand_lo_y = t_lo_y | jnp.left_shift(jnp.int32(1), b)
        cand_lo = (cand_lo_y ^ 0x8000).astype(jnp.int16)
        cnt = cnt_gt_hi + _count16(eq_hi & (lo_b >= cand_lo))
        return jnp.where(cnt >= TOPK, cand_lo_y, t_lo_y)

    t_lo_y = jax.lax.fori_loop(0, 6, bit_body_lo, jnp.zeros((R, 1), jnp.int32))
    t_y = jnp.left_shift(t_y16, 16) | t_lo_y
    t_key = t_y ^ jnp.int32(INT_MIN)

    up_y = t_y + 1024
    wrap = (t_y >= -1024) & (t_y < 0)
    up_key = up_y ^ jnp.int32(INT_MIN)
    gt = (key >= up_key) & jnp.logical_not(wrap)
    band = (key >= t_key) & jnp.logical_not(gt)
    cnt_gt = _count32(gt)
    cnt_band = _count32(band)
    col = jax.lax.broadcasted_iota(jnp.int32, (R, N), 1)

    skey = jnp.where(band, ((key & 0x3FF) << 12) | (N - 1 - col), -1)

    surplus = (cnt_gt + cnt_band) > TOPK
    cnt0 = jnp.where(surplus, cnt_gt, jnp.float32(TOPK))
    big = jnp.int32(0x7FFFFFFF)
    t0 = jnp.where(surplus, big, 0)

    def cond_fn(carry):
        cnt, _ = carry
        return jnp.any(cnt < TOPK)

    def body_fn(carry):
        cnt, t_last = carry
        m = jnp.max(jnp.where(skey < t_last, skey, -1), axis=1, keepdims=True)
        upd = cnt < TOPK
        return jnp.where(upd, cnt + 1, cnt), jnp.where(upd, m, t_last)

    _, t_fin = jax.lax.while_loop(cond_fn, body_fn, (cnt0, t0))

    sel = gt | (skey >= t_fin)
    w_sel = jnp.where(sel, w, 0.0)
    rowsum = jnp.sum(w_sel, axis=1, keepdims=True)
    out_ref[...] = w_sel / (rowsum + 1e-8)


def _topk_norm(w, row_block=256):
    grid = N // row_block
    return pl.pallas_call(
        _topk_norm_kernel,
        grid=(grid,),
        in_specs=[pl.BlockSpec((row_block, N), lambda i: (i, 0))],
        out_specs=pl.BlockSpec((row_block, N), lambda i: (i, 0)),
        out_shape=jax.ShapeDtypeStruct((N, N), jnp.float32),
    )(w)


def _mm_bias_kernel(x_ref, w_ref, b_ref, o_ref):
    o_ref[...] = (
        jnp.dot(x_ref[...], w_ref[...], preferred_element_type=jnp.float32)
        + b_ref[...]
    )


def _mm_bias(x, w, b, row_block=512):
    grid = x.shape[0] // row_block
    return pl.pallas_call(
        _mm_bias_kernel,
        grid=(grid,),
        in_specs=[
            pl.BlockSpec((row_block, x.shape[1]), lambda i: (i, 0)),
            pl.BlockSpec(w.shape, lambda i: (0, 0)),
            pl.BlockSpec(b.shape, lambda i: (0, 0)),
        ],
        out_specs=pl.BlockSpec((row_block, w.shape[1]), lambda i: (i, 0)),
        out_shape=jax.ShapeDtypeStruct((x.shape[0], w.shape[1]), jnp.float32),
    )(x, w, b)


def _agg_out_gate_kernel(wn_ref, v_ref, h_ref, wo_ref, bo_ref, wgt_ref, bg_ref, o_ref):
    msg = jnp.dot(wn_ref[...], v_ref[...], preferred_element_type=jnp.float32)
    out = jnp.dot(msg, wo_ref[...], preferred_element_type=jnp.float32) + bo_ref[...]
    hv = h_ref[...]
    logit = jnp.sum(hv * wgt_ref[...], axis=1, keepdims=True) + bg_ref[...]
    alpha = jax.nn.sigmoid(logit)
    o_ref[...] = alpha * hv + (1.0 - alpha) * out


def _agg_out_gate(w_norm, v, h, wo, bo2, wgt, bg2, row_block=256):
    grid = N // row_block
    return pl.pallas_call(
        _agg_out_gate_kernel,
        grid=(grid,),
        in_specs=[
            pl.BlockSpec((row_block, N), lambda i: (i, 0)),
            pl.BlockSpec((N, D), lambda i: (0, 0)),
            pl.BlockSpec((row_block, D), lambda i: (i, 0)),
            pl.BlockSpec((D, D), lambda i: (0, 0)),
            pl.BlockSpec((1, D), lambda i: (0, 0)),
            pl.BlockSpec((1, D), lambda i: (0, 0)),
            pl.BlockSpec((1, 1), lambda i: (0, 0)),
        ],
        out_specs=pl.BlockSpec((row_block, D), lambda i: (i, 0)),
        out_shape=jax.ShapeDtypeStruct((N, D), jnp.float32),
    )(w_norm, v, h, wo, bo2, wgt, bg2)


def kernel(h, w, Wv0, bv0, Wo0, bo0, Wv1, bv1, Wo1, bo1, Wg, bg):
    w_norm = _topk_norm(w)
    wgt = Wg.reshape(1, D)
    bg2 = bg.reshape(1, 1)
    for Wv, bv, Wo, bo in ((Wv0, bv0, Wo0, bo0), (Wv1, bv1, Wo1, bo1)):
        v = _mm_bias(h, Wv, bv.reshape(1, D))
        h = _agg_out_gate(w_norm, v, h, Wo, bo.reshape(1, D), wgt, bg2)
    return h


# speedup vs baseline: 1.9816x; 1.9816x over previous
import functools

import jax
import jax.numpy as jnp
from jax.experimental import pallas as pl
from jax.experimental.pallas import tpu as pltpu

N = 4096
D = 512
TOPK = 32
INT_MIN = -2147483648


def _topk_norm_kernel(w_ref, out_ref):
    R = w_ref.shape[0]
    w = w_ref[...]
    key = jax.lax.bitcast_convert_type(w, jnp.int32)
    key = key ^ (jax.lax.shift_right_arithmetic(key, 31) & jnp.int32(0x7FFFFFFF))
    hi = jax.lax.shift_right_arithmetic(key, 16).astype(jnp.int16)

    def _count16(mask):
        mbf = jnp.where(mask, jnp.bfloat16(1), jnp.bfloat16(0))
        half = mbf[:, : N // 2] + mbf[:, N // 2 :]
        quarter = half[:, : N // 4] + half[:, N // 4 :]
        return jnp.sum(quarter.astype(jnp.float32), axis=1, keepdims=True)

    def _count32(mask):
        mf = jnp.where(mask, 1.0, 0.0)
        return jnp.sum(mf, axis=1, keepdims=True)

    def bit_body16(i, t_y):
        b = 15 - i
        cand_y = t_y | jnp.left_shift(jnp.int32(1), b)
        cand_key = (cand_y ^ 0x8000).astype(jnp.int16)
        cnt = _count16(hi >= cand_key)
        return jnp.where(cnt >= TOPK, cand_y, t_y)

    t_y16 = jax.lax.fori_loop(0, 16, bit_body16, jnp.zeros((R, 1), jnp.int32))

    t16 = (t_y16 ^ 0x8000).astype(jnp.int16)
    gt_hi = hi > t16
    eq_hi = hi == t16
    cnt_gt_hi = _count16(gt_hi)
    lo_b = ((key & 0xFFFF) ^ 0x8000).astype(jnp.int16)

    def bit_body_lo(i, t_lo_y):
        b = 15 - i
        cand_lo_y = t_lo_y | jnp.left_shift(jnp.int32(1), b)
        cand_lo = (cand_lo_y ^ 0x8000).astype(jnp.int16)
        cnt = cnt_gt_hi + _count16(eq_hi & (lo_b >= cand_lo))
        return jnp.where(cnt >= TOPK, cand_lo_y, t_lo_y)

    t_lo_y = jax.lax.fori_loop(0, 6, bit_body_lo, jnp.zeros((R, 1), jnp.int32))
    t_y = jnp.left_shift(t_y16, 16) | t_lo_y
    t_key = t_y ^ jnp.int32(INT_MIN)

    up_y = t_y + 1024
    wrap = (t_y >= -1024) & (t_y < 0)
    up_key = up_y ^ jnp.int32(INT_MIN)
    gt = (key >= up_key) & jnp.logical_not(wrap)
    band = (key >= t_key) & jnp.logical_not(gt)
    cnt_gt = _count32(gt)
    cnt_band = _count32(band)
    col = jax.lax.broadcasted_iota(jnp.int32, (R, N), 1)

    skey = jnp.where(band, ((key & 0x3FF) << 12) | (N - 1 - col), -1)

    surplus = (cnt_gt + cnt_band) > TOPK
    cnt0 = jnp.where(surplus, cnt_gt, jnp.float32(TOPK))
    big = jnp.int32(0x7FFFFFFF)
    t0 = jnp.where(surplus, big, 0)

    def cond_fn(carry):
        cnt, _ = carry
        return jnp.any(cnt < TOPK)

    def body_fn(carry):
        cnt, t_last = carry
        m = jnp.max(jnp.where(skey < t_last, skey, -1), axis=1, keepdims=True)
        upd = cnt < TOPK
        return jnp.where(upd, cnt + 1, cnt), jnp.where(upd, m, t_last)

    _, t_fin = jax.lax.while_loop(cond_fn, body_fn, (cnt0, t0))

    sel = gt | (skey >= t_fin)
    w_sel = jnp.where(sel, w, 0.0)
    rowsum = jnp.sum(w_sel, axis=1, keepdims=True)
    out_ref[...] = w_sel / (rowsum + 1e-8)


def _topk_norm(w, row_block=256):
    grid = N // row_block
    return pl.pallas_call(
        _topk_norm_kernel,
        grid=(grid,),
        in_specs=[pl.BlockSpec((row_block, N), lambda i: (i, 0))],
        out_specs=pl.BlockSpec((row_block, N), lambda i: (i, 0)),
        out_shape=jax.ShapeDtypeStruct((N, N), jnp.float32),
    )(w)


def _mm_bias_kernel(x_ref, w_ref, b_ref, o_ref):
    o_ref[...] = (
        jnp.dot(x_ref[...], w_ref[...], preferred_element_type=jnp.float32)
        + b_ref[...]
    )


def _mm_bias(x, w, b, row_block=512):
    grid = x.shape[0] // row_block
    return pl.pallas_call(
        _mm_bias_kernel,
        grid=(grid,),
        in_specs=[
            pl.BlockSpec((row_block, x.shape[1]), lambda i: (i, 0)),
            pl.BlockSpec(w.shape, lambda i: (0, 0)),
            pl.BlockSpec(b.shape, lambda i: (0, 0)),
        ],
        out_specs=pl.BlockSpec((row_block, w.shape[1]), lambda i: (i, 0)),
        out_shape=jax.ShapeDtypeStruct((x.shape[0], w.shape[1]), jnp.float32),
    )(x, w, b)


def _agg_out_gate_kernel(wn_ref, v_ref, h_ref, wo_ref, bo_ref, wgt_ref, bg_ref, o_ref):
    msg = jnp.dot(wn_ref[...], v_ref[...], preferred_element_type=jnp.float32)
    out = jnp.dot(msg, wo_ref[...], preferred_element_type=jnp.float32) + bo_ref[...]
    hv = h_ref[...]
    logit = jnp.sum(hv * wgt_ref[...], axis=1, keepdims=True) + bg_ref[...]
    alpha = jax.nn.sigmoid(logit)
    o_ref[...] = alpha * hv + (1.0 - alpha) * out


def _agg_out_gate(w_norm, v, h, wo, bo2, wgt, bg2, row_block=256):
    grid = N // row_block
    return pl.pallas_call(
        _agg_out_gate_kernel,
        grid=(grid,),
        in_specs=[
            pl.BlockSpec((row_block, N), lambda i: (i, 0)),
            pl.BlockSpec((N, D), lambda i: (0, 0)),
            pl.BlockSpec((row_block, D), lambda i: (i, 0)),
            pl.BlockSpec((D, D), lambda i: (0, 0)),
            pl.BlockSpec((1, D), lambda i: (0, 0)),
            pl.BlockSpec((1, D), lambda i: (0, 0)),
            pl.BlockSpec((1, 1), lambda i: (0, 0)),
        ],
        out_specs=pl.BlockSpec((row_block, D), lambda i: (i, 0)),
        out_shape=jax.ShapeDtypeStruct((N, D), jnp.float32),
    )(w_norm, v, h, wo, bo2, wgt, bg2)


def kernel(h, w, Wv0, bv0, Wo0, bo0, Wv1, bv1, Wo1, bo1, Wg, bg):
    w_norm = _topk_norm(w)
    wgt = Wg.reshape(1, D)
    bg2 = bg.reshape(1, 1)
    for Wv, bv, Wo, bo in ((Wv0, bv0, Wo0, bo0), (Wv1, bv1, Wo1, bo1)):
        v = _mm_bias(h, Wv, bv.reshape(1, D))
        h = _agg_out_gate(w_norm, v, h, Wo, bo.reshape(1, D), wgt, bg2)
    return h
